# baseline (speedup 1.0000x reference)
import functools

import jax
import jax.numpy as jnp
from jax.experimental import pallas as pl
from jax.experimental.pallas import tpu as pltpu

H = 1024
I = 512
E = 8
K = 2
T = 2048
TB = 1024
NEG = -1e30


def _silu(x):
    return x * jax.nn.sigmoid(x)


def _gate_mat(x, gw, b, rows):
    logits = jax.lax.dot_general(x, gw, (((1,), (1,)), ((), ())),
                                 preferred_element_type=jnp.float32)
    scores = jax.nn.sigmoid(logits)
    routing = scores + b
    iota = jax.lax.broadcasted_iota(jnp.int32, (rows, E), 1)
    m1 = jnp.max(routing, axis=1, keepdims=True)
    a1 = jnp.min(jnp.where(routing == m1, iota, E), axis=1, keepdims=True)
    routing2 = jnp.where(iota == a1, NEG, routing)
    m2 = jnp.max(routing2, axis=1, keepdims=True)
    a2 = jnp.min(jnp.where(routing2 == m2, iota, E), axis=1, keepdims=True)
    s1 = jnp.sum(jnp.where(iota == a1, scores, 0.0), axis=1, keepdims=True)
    s2 = jnp.sum(jnp.where(iota == a2, scores, 0.0), axis=1, keepdims=True)
    denom = s1 + s2
    return (jnp.where(iota == a1, s1, 0.0)
            + jnp.where(iota == a2, s2, 0.0)) / denom


def _moe_body(x_ref, gw_ref, b_ref, w1_ref, w3_ref, w2_ref,
              sw1_ref, sw3_ref, sw2_ref, out_ref, gate_scr):
    e = pl.program_id(0)

    @pl.when(e == 0)
    def _router():
        gate_scr[...] = _gate_mat(x_ref[...], gw_ref[...], b_ref[...], T)
        out_ref[...] = jnp.zeros_like(out_ref)

    lane = jax.lax.broadcasted_iota(jnp.int32, (TB, E), 1)

    @pl.when(e < E)
    def _routed():
        for j in range(T // TB):
            sl = pl.ds(j * TB, TB)
            xj = x_ref[sl, :]
            h1 = jax.lax.dot_general(xj, w1_ref[0], (((1,), (1,)), ((), ())),
                                     preferred_element_type=jnp.float32)
            h3 = jax.lax.dot_general(xj, w3_ref[0], (((1,), (1,)), ((), ())),
                                     preferred_element_type=jnp.float32)
            gate_col = jnp.sum(
                jnp.where(lane == e, gate_scr[sl, :], 0.0), axis=1,
                keepdims=True)
            act = _silu(h1) * h3 * gate_col
            y = jax.lax.dot_general(act, w2_ref[0], (((1,), (1,)), ((), ())),
                                    preferred_element_type=jnp.float32)
            out_ref[sl, :] += y

    @pl.when(e == E)
    def _shared():
        for j in range(T // TB):
            sl = pl.ds(j * TB, TB)
            xj = x_ref[sl, :]
            h1 = jax.lax.dot_general(xj, sw1_ref[...], (((1,), (1,)), ((), ())),
                                     preferred_element_type=jnp.float32)
            h3 = jax.lax.dot_general(xj, sw3_ref[...], (((1,), (1,)), ((), ())),
                                     preferred_element_type=jnp.float32)
            act = _silu(h1) * h3
            y = jax.lax.dot_general(act, sw2_ref[...], (((1,), (1,)), ((), ())),
                                    preferred_element_type=jnp.float32)
            out_ref[sl, :] += y


def kernel(hidden_states, gate_w, bias, W1, W2, W3, SW1, SW2, SW3):
    orig_shape = hidden_states.shape
    x = hidden_states.reshape(T, H)
    bias2 = bias.reshape(1, E)

    out = pl.pallas_call(
        _moe_body,
        grid=(E + 1,),
        in_specs=[
            pl.BlockSpec((T, H), lambda e: (0, 0)),
            pl.BlockSpec((E, H), lambda e: (0, 0)),
            pl.BlockSpec((1, E), lambda e: (0, 0)),
            pl.BlockSpec((1, I, H), lambda e: (jnp.minimum(e, E - 1), 0, 0)),
            pl.BlockSpec((1, I, H), lambda e: (jnp.minimum(e, E - 1), 0, 0)),
            pl.BlockSpec((1, H, I), lambda e: (jnp.minimum(e, E - 1), 0, 0)),
            pl.BlockSpec((I, H), lambda e: (0, 0)),
            pl.BlockSpec((I, H), lambda e: (0, 0)),
            pl.BlockSpec((H, I), lambda e: (0, 0)),
        ],
        out_specs=pl.BlockSpec((T, H), lambda e: (0, 0)),
        out_shape=jax.ShapeDtypeStruct((T, H), jnp.float32),
        scratch_shapes=[pltpu.VMEM((T, E), jnp.float32)],
    )(x, gate_w, bias2, W1, W3, W2, SW1, SW3, SW2)
    return out.reshape(orig_shape)


# speedup vs baseline: 1.0058x; 1.0058x over previous
import functools

import jax
import jax.numpy as jnp
from jax.experimental import pallas as pl
from jax.experimental.pallas import tpu as pltpu

H = 1024
I = 512
E = 8
K = 2
T = 2048
TB = 1024
NEG = -1e30


def _silu(x):
    return x * jax.nn.sigmoid(x)


def _gate_mat(x, gw, b, rows):
    logits = jax.lax.dot_general(x, gw, (((1,), (1,)), ((), ())),
                                 preferred_element_type=jnp.float32)
    scores = jax.nn.sigmoid(logits)
    routing = scores + b
    iota = jax.lax.broadcasted_iota(jnp.int32, (rows, E), 1)
    m1 = jnp.max(routing, axis=1, keepdims=True)
    a1 = jnp.min(jnp.where(routing == m1, iota, E), axis=1, keepdims=True)
    routing2 = jnp.where(iota == a1, NEG, routing)
    m2 = jnp.max(routing2, axis=1, keepdims=True)
    a2 = jnp.min(jnp.where(routing2 == m2, iota, E), axis=1, keepdims=True)
    s1 = jnp.sum(jnp.where(iota == a1, scores, 0.0), axis=1, keepdims=True)
    s2 = jnp.sum(jnp.where(iota == a2, scores, 0.0), axis=1, keepdims=True)
    denom = s1 + s2
    return (jnp.where(iota == a1, s1, 0.0)
            + jnp.where(iota == a2, s2, 0.0)) / denom


def _moe_body(x_ref, gw_ref, b_ref, w1_ref, w3_ref, w2_ref,
              sw1_ref, sw3_ref, sw2_ref, out_ref, gate_scr, xbf_scr):
    e = pl.program_id(0)

    @pl.when(e == 0)
    def _router():
        gate_scr[...] = _gate_mat(x_ref[...], gw_ref[...], b_ref[...], T)
        xbf_scr[...] = x_ref[...].astype(jnp.bfloat16)
        out_ref[...] = jnp.zeros_like(out_ref)

    lane = jax.lax.broadcasted_iota(jnp.int32, (TB, E), 1)

    @pl.when(e < E)
    def _routed():
        w1 = w1_ref[0].astype(jnp.bfloat16)
        w3 = w3_ref[0].astype(jnp.bfloat16)
        w2 = w2_ref[0].astype(jnp.bfloat16)
        for j in range(T // TB):
            sl = pl.ds(j * TB, TB)
            xj = xbf_scr[sl, :]
            h1 = jax.lax.dot_general(xj, w1, (((1,), (1,)), ((), ())),
                                     preferred_element_type=jnp.float32)
            h3 = jax.lax.dot_general(xj, w3, (((1,), (1,)), ((), ())),
                                     preferred_element_type=jnp.float32)
            act = (_silu(h1) * h3).astype(jnp.bfloat16)
            y = jax.lax.dot_general(act, w2, (((1,), (1,)), ((), ())),
                                    preferred_element_type=jnp.float32)
            gate_col = jnp.sum(
                jnp.where(lane == e, gate_scr[sl, :], 0.0), axis=1,
                keepdims=True)
            out_ref[sl, :] += y * gate_col

    @pl.when(e == E)
    def _shared():
        w1 = sw1_ref[...].astype(jnp.bfloat16)
        w3 = sw3_ref[...].astype(jnp.bfloat16)
        w2 = sw2_ref[...].astype(jnp.bfloat16)
        for j in range(T // TB):
            sl = pl.ds(j * TB, TB)
            xj = xbf_scr[sl, :]
            h1 = jax.lax.dot_general(xj, w1, (((1,), (1,)), ((), ())),
                                     preferred_element_type=jnp.float32)
            h3 = jax.lax.dot_general(xj, w3, (((1,), (1,)), ((), ())),
                                     preferred_element_type=jnp.float32)
            act = (_silu(h1) * h3).astype(jnp.bfloat16)
            y = jax.lax.dot_general(act, w2, (((1,), (1,)), ((), ())),
                                    preferred_element_type=jnp.float32)
            out_ref[sl, :] += y


def kernel(hidden_states, gate_w, bias, W1, W2, W3, SW1, SW2, SW3):
    orig_shape = hidden_states.shape
    x = hidden_states.reshape(T, H)
    bias2 = bias.reshape(1, E)

    out = pl.pallas_call(
        _moe_body,
        grid=(E + 1,),
        in_specs=[
            pl.BlockSpec((T, H), lambda e: (0, 0)),
            pl.BlockSpec((E, H), lambda e: (0, 0)),
            pl.BlockSpec((1, E), lambda e: (0, 0)),
            pl.BlockSpec((1, I, H), lambda e: (jnp.minimum(e, E - 1), 0, 0)),
            pl.BlockSpec((1, I, H), lambda e: (jnp.minimum(e, E - 1), 0, 0)),
            pl.BlockSpec((1, H, I), lambda e: (jnp.minimum(e, E - 1), 0, 0)),
            pl.BlockSpec((I, H), lambda e: (0, 0)),
            pl.BlockSpec((I, H), lambda e: (0, 0)),
            pl.BlockSpec((H, I), lambda e: (0, 0)),
        ],
        out_specs=pl.BlockSpec((T, H), lambda e: (0, 0)),
        out_shape=jax.ShapeDtypeStruct((T, H), jnp.float32),
        scratch_shapes=[pltpu.VMEM((T, E), jnp.float32),
                        pltpu.VMEM((T, H), jnp.bfloat16)],
    )(x, gate_w, bias2, W1, W3, W2, SW1, SW3, SW2)
    return out.reshape(orig_shape)


# speedup vs baseline: 1.0194x; 1.0135x over previous
import functools

import jax
import jax.numpy as jnp
from jax.experimental import pallas as pl
from jax.experimental.pallas import tpu as pltpu

H = 1024
I = 512
E = 8
K = 2
T = 2048
TB = 1024
NEG = -1e30


def _silu(x):
    return x * jax.nn.sigmoid(x)


def _gate_mat(x, gw, b, rows):
    logits = jax.lax.dot_general(x, gw, (((1,), (1,)), ((), ())),
                                 preferred_element_type=jnp.float32)
    scores = jax.nn.sigmoid(logits)
    routing = scores + b
    iota = jax.lax.broadcasted_iota(jnp.int32, (rows, E), 1)
    m1 = jnp.max(routing, axis=1, keepdims=True)
    a1 = jnp.min(jnp.where(routing == m1, iota, E), axis=1, keepdims=True)
    routing2 = jnp.where(iota == a1, NEG, routing)
    m2 = jnp.max(routing2, axis=1, keepdims=True)
    a2 = jnp.min(jnp.where(routing2 == m2, iota, E), axis=1, keepdims=True)
    s1 = jnp.sum(jnp.where(iota == a1, scores, 0.0), axis=1, keepdims=True)
    s2 = jnp.sum(jnp.where(iota == a2, scores, 0.0), axis=1, keepdims=True)
    denom = s1 + s2
    return (jnp.where(iota == a1, s1, 0.0)
            + jnp.where(iota == a2, s2, 0.0)) / denom


def _moe_body(x_ref, gw_ref, b_ref, w1_ref, w3_ref, w2_ref,
              sw1_ref, sw3_ref, sw2_ref, out_ref, gate_scr):
    e = pl.program_id(0)

    @pl.when(e == 0)
    def _router():
        gate_scr[...] = _gate_mat(x_ref[...], gw_ref[...], b_ref[...], T)
        out_ref[...] = jnp.zeros_like(out_ref)

    lane = jax.lax.broadcasted_iota(jnp.int32, (TB, E), 1)

    @pl.when(e < E)
    def _routed():
        for j in range(T // TB):
            sl = pl.ds(j * TB, TB)
            xj = x_ref[sl, :]
            h1 = jax.lax.dot_general(xj, w1_ref[0], (((1,), (1,)), ((), ())),
                                     preferred_element_type=jnp.float32)
            h3 = jax.lax.dot_general(xj, w3_ref[0], (((1,), (1,)), ((), ())),
                                     preferred_element_type=jnp.float32)
            act = _silu(h1) * h3
            y = jax.lax.dot_general(act, w2_ref[0], (((1,), (1,)), ((), ())),
                                    preferred_element_type=jnp.float32)
            gate_col = jnp.sum(
                jnp.where(lane == e, gate_scr[sl, :], 0.0), axis=1,
                keepdims=True)
            out_ref[sl, :] += y * gate_col

    @pl.when(e == E)
    def _shared():
        for j in range(T // TB):
            sl = pl.ds(j * TB, TB)
            xj = x_ref[sl, :]
            h1 = jax.lax.dot_general(xj, sw1_ref[...], (((1,), (1,)), ((), ())),
                                     preferred_element_type=jnp.float32)
            h3 = jax.lax.dot_general(xj, sw3_ref[...], (((1,), (1,)), ((), ())),
                                     preferred_element_type=jnp.float32)
            act = _silu(h1) * h3
            y = jax.lax.dot_general(act, sw2_ref[...], (((1,), (1,)), ((), ())),
                                    preferred_element_type=jnp.float32)
            out_ref[sl, :] += y


def kernel(hidden_states, gate_w, bias, W1, W2, W3, SW1, SW2, SW3):
    orig_shape = hidden_states.shape
    x = hidden_states.reshape(T, H)
    bias2 = bias.reshape(1, E)

    out = pl.pallas_call(
        _moe_body,
        grid=(E + 1,),
        in_specs=[
            pl.BlockSpec((T, H), lambda e: (0, 0)),
            pl.BlockSpec((E, H), lambda e: (0, 0)),
            pl.BlockSpec((1, E), lambda e: (0, 0)),
            pl.BlockSpec((1, I, H), lambda e: (jnp.minimum(e, E - 1), 0, 0)),
            pl.BlockSpec((1, I, H), lambda e: (jnp.minimum(e, E - 1), 0, 0)),
            pl.BlockSpec((1, H, I), lambda e: (jnp.minimum(e, E - 1), 0, 0)),
            pl.BlockSpec((I, H), lambda e: (0, 0)),
            pl.BlockSpec((I, H), lambda e: (0, 0)),
            pl.BlockSpec((H, I), lambda e: (0, 0)),
        ],
        out_specs=pl.BlockSpec((T, H), lambda e: (0, 0)),
        out_shape=jax.ShapeDtypeStruct((T, H), jnp.float32),
        scratch_shapes=[pltpu.VMEM((T, E), jnp.float32)],
    )(x, gate_w, bias2, W1, W3, W2, SW1, SW3, SW2)
    return out.reshape(orig_shape)


# speedup vs baseline: 1.0262x; 1.0067x over previous
import functools

import jax
import jax.numpy as jnp
from jax.experimental import pallas as pl
from jax.experimental.pallas import tpu as pltpu

H = 1024
I = 512
E = 8
K = 2
T = 2048
TB = 1024
NEG = -1e30


def _silu(x):
    return x * jax.nn.sigmoid(x)


def _gate_mat(x, gw, b, rows):
    logits = jax.lax.dot_general(x, gw, (((1,), (1,)), ((), ())),
                                 preferred_element_type=jnp.float32)
    scores = jax.nn.sigmoid(logits)
    routing = scores + b
    iota = jax.lax.broadcasted_iota(jnp.int32, (rows, E), 1)
    m1 = jnp.max(routing, axis=1, keepdims=True)
    a1 = jnp.min(jnp.where(routing == m1, iota, E), axis=1, keepdims=True)
    routing2 = jnp.where(iota == a1, NEG, routing)
    m2 = jnp.max(routing2, axis=1, keepdims=True)
    a2 = jnp.min(jnp.where(routing2 == m2, iota, E), axis=1, keepdims=True)
    s1 = jnp.sum(jnp.where(iota == a1, scores, 0.0), axis=1, keepdims=True)
    s2 = jnp.sum(jnp.where(iota == a2, scores, 0.0), axis=1, keepdims=True)
    denom = s1 + s2
    return (jnp.where(iota == a1, s1, 0.0)
            + jnp.where(iota == a2, s2, 0.0)) / denom


def _moe_body(x_ref, gw_ref, b_ref, w1_ref, w3_ref, w2_ref,
              sw1_ref, sw3_ref, sw2_ref, out_ref, gate_scr):
    e = pl.program_id(0)

    @pl.when(e == 0)
    def _router():
        gate_scr[...] = _gate_mat(x_ref[...], gw_ref[...], b_ref[...], T)

    lane = jax.lax.broadcasted_iota(jnp.int32, (TB, E), 1)

    def _routed_step(j, first):
        sl = pl.ds(j * TB, TB)
        xj = x_ref[sl, :]
        h1 = jax.lax.dot_general(xj, w1_ref[0], (((1,), (1,)), ((), ())),
                                 preferred_element_type=jnp.float32)
        h3 = jax.lax.dot_general(xj, w3_ref[0], (((1,), (1,)), ((), ())),
                                 preferred_element_type=jnp.float32)
        act = _silu(h1) * h3
        y = jax.lax.dot_general(act, w2_ref[0], (((1,), (1,)), ((), ())),
                                preferred_element_type=jnp.float32)
        gate_col = jnp.sum(
            jnp.where(lane == e, gate_scr[sl, :], 0.0), axis=1,
            keepdims=True)
        if first:
            out_ref[sl, :] = y * gate_col
        else:
            out_ref[sl, :] += y * gate_col

    @pl.when(e == 0)
    def _routed_first():
        for j in range(T // TB):
            _routed_step(j, True)

    @pl.when((e > 0) & (e < E))
    def _routed():
        for j in range(T // TB):
            _routed_step(j, False)

    @pl.when(e == E)
    def _shared():
        for j in range(T // TB):
            sl = pl.ds(j * TB, TB)
            xj = x_ref[sl, :]
            h1 = jax.lax.dot_general(xj, sw1_ref[...], (((1,), (1,)), ((), ())),
                                     preferred_element_type=jnp.float32)
            h3 = jax.lax.dot_general(xj, sw3_ref[...], (((1,), (1,)), ((), ())),
                                     preferred_element_type=jnp.float32)
            act = _silu(h1) * h3
            y = jax.lax.dot_general(act, sw2_ref[...], (((1,), (1,)), ((), ())),
                                    preferred_element_type=jnp.float32)
            out_ref[sl, :] += y


def kernel(hidden_states, gate_w, bias, W1, W2, W3, SW1, SW2, SW3):
    orig_shape = hidden_states.shape
    x = hidden_states.reshape(T, H)
    bias2 = bias.reshape(1, E)

    out = pl.pallas_call(
        _moe_body,
        grid=(E + 1,),
        in_specs=[
            pl.BlockSpec((T, H), lambda e: (0, 0)),
            pl.BlockSpec((E, H), lambda e: (0, 0)),
            pl.BlockSpec((1, E), lambda e: (0, 0)),
            pl.BlockSpec((1, I, H), lambda e: (jnp.minimum(e, E - 1), 0, 0)),
            pl.BlockSpec((1, I, H), lambda e: (jnp.minimum(e, E - 1), 0, 0)),
            pl.BlockSpec((1, H, I), lambda e: (jnp.minimum(e, E - 1), 0, 0)),
            pl.BlockSpec((I, H), lambda e: (0, 0)),
            pl.BlockSpec((I, H), lambda e: (0, 0)),
            pl.BlockSpec((H, I), lambda e: (0, 0)),
        ],
        out_specs=pl.BlockSpec((T, H), lambda e: (0, 0)),
        out_shape=jax.ShapeDtypeStruct((T, H), jnp.float32),
        scratch_shapes=[pltpu.VMEM((T, E), jnp.float32)],
    )(x, gate_w, bias2, W1, W3, W2, SW1, SW3, SW2)
    return out.reshape(orig_shape)
